# baseline (speedup 1.0000x reference)
import functools

import jax
import jax.numpy as jnp
from jax import lax
from jax.experimental import pallas as pl
from jax.experimental.pallas import tpu as pltpu
from jax.experimental.pallas import tpu_sc as plsc

N = 10000
E = 320000
EPRIME = E + N
D = 128
H = 128
NB = 64

NPAD = 10240
CH = 80
NTILES = 32
NBUF = 4
NCHUNK = 136
EARR = NTILES * CH * NCHUNK
ROWS_PER_TILE = NPAD // 16

_f32 = jnp.float32
_i32 = jnp.int32


def _tc_feats_body(x_ref, w_ref, att_ref, ea_ref, h_ref, sd_ref, ss_ref, eaw_ref):
    h = jnp.dot(x_ref[...], w_ref[...], preferred_element_type=_f32)
    h_ref[...] = h
    att = att_ref[0, 0, :]
    att_d = att[0:H].reshape(H, 1)
    att_s = att[H:2 * H].reshape(H, 1)
    sd_ref[...] = jnp.dot(h, att_d, preferred_element_type=_f32)
    ss_ref[...] = jnp.dot(h, att_s, preferred_element_type=_f32)
    eaw_ref[...] = ea_ref[...] * att_ref[0, 0, 2 * H]


def _tc_feats(x_pad, w, att, ea2d):
    return pl.pallas_call(
        _tc_feats_body,
        out_shape=[
            jax.ShapeDtypeStruct((NPAD, D), _f32),
            jax.ShapeDtypeStruct((NPAD, 1), _f32),
            jax.ShapeDtypeStruct((NPAD, 1), _f32),
            jax.ShapeDtypeStruct((EARR // 128, 128), _f32),
        ],
    )(x_pad, w, att, ea2d)


def _tc_combine_body(ag_ref, den_ref, b_ref, w_ref, att_ref, ea_ref,
                     h_ref, sd_ref, ss_ref, eaw_ref):
    a = ag_ref[0] + ag_ref[1]
    dsum = den_ref[0] + den_ref[1]
    hin = jnp.maximum(a / (dsum + 1e-16) + b_ref[...][None, :], 0.0)
    h = jnp.dot(hin, w_ref[...], preferred_element_type=_f32)
    h_ref[...] = h
    att = att_ref[0, 0, :]
    att_d = att[0:H].reshape(H, 1)
    att_s = att[H:2 * H].reshape(H, 1)
    sd_ref[...] = jnp.dot(h, att_d, preferred_element_type=_f32)
    ss_ref[...] = jnp.dot(h, att_s, preferred_element_type=_f32)
    eaw_ref[...] = ea_ref[...] * att_ref[0, 0, 2 * H]


def _tc_combine(aggr, den3, b, w, att, ea2d):
    return pl.pallas_call(
        _tc_combine_body,
        out_shape=[
            jax.ShapeDtypeStruct((NPAD, D), _f32),
            jax.ShapeDtypeStruct((NPAD, 1), _f32),
            jax.ShapeDtypeStruct((NPAD, 1), _f32),
            jax.ShapeDtypeStruct((EARR // 128, 128), _f32),
        ],
    )(aggr, den3, b, w, att, ea2d)


def _tc_final_body(ag_ref, den_ref, b_ref, batch_ref, wf_ref, bf_ref, y_ref):
    a = ag_ref[0] + ag_ref[1]
    dsum = den_ref[0] + den_ref[1]
    h = jnp.maximum(a / (dsum + 1e-16) + b_ref[...][None, :], 0.0)
    ids = lax.broadcasted_iota(_i32, (1, NB), 1)
    oh = (batch_ref[...] == ids).astype(_f32)
    pooled = lax.dot_general(oh, h, (((0,), (0,)), ((), ())),
                             preferred_element_type=_f32)
    y_ref[...] = jnp.dot(pooled, wf_ref[...], preferred_element_type=_f32) + bf_ref[0]


def _tc_final(aggr, den3, b, batchcol, wf, bf):
    return pl.pallas_call(
        _tc_final_body,
        out_shape=jax.ShapeDtypeStruct((NB, 1), _f32),
    )(aggr, den3, b, batchcol, wf, bf)


def _lane_bcast(v, lane):
    idx = jnp.full((16, 1), lane, _i32)
    dnums = lax.GatherDimensionNumbers(offset_dims=(), collapsed_slice_dims=(0,),
                                       start_index_map=(0,))
    return lax.gather(v, idx, dnums, (1,),
                      mode=lax.GatherScatterMode.PROMISE_IN_BOUNDS)


def _sc_edge_body(src_hbm, dst_hbm, eaw_hbm, sd_hbm, ss_hbm, h_hbm,
                  zrow_hbm, zvec_hbm,
                  aggr_out, den_out,
                  srcv, dstv, eav, ev, sdg, ssg, rows,
                  aggr_sh, den_sh, gsem, asem):
    cid = lax.axis_index("c")
    sid = lax.axis_index("s")
    wid = cid * 16 + sid

    pltpu.sync_copy(zrow_hbm, aggr_sh.at[pl.ds(sid * ROWS_PER_TILE, ROWS_PER_TILE)])
    pltpu.sync_copy(zvec_hbm, den_sh.at[pl.ds(sid * ROWS_PER_TILE, ROWS_PER_TILE)])
    plsc.subcore_barrier()

    def fetch(c, b):
        base = (wid * NCHUNK + c) * CH
        pltpu.sync_copy(src_hbm.at[pl.ds(base, CH)], srcv.at[b])
        pltpu.sync_copy(dst_hbm.at[pl.ds(base, CH)], dstv.at[b])
        pltpu.sync_copy(eaw_hbm.at[pl.ds(base, CH)], eav.at[b])
        pltpu.async_copy(h_hbm.at[srcv.at[b]], rows.at[b], gsem.at[b])
        pltpu.async_copy(sd_hbm.at[dstv.at[b]], sdg.at[b], gsem.at[b])
        pltpu.async_copy(ss_hbm.at[srcv.at[b]], ssg.at[b], gsem.at[b])

    def compute_and_scatter(c, b):
        base = (wid * NCHUNK + c) * CH
        sv, dv, av, evb, rb = srcv.at[b], dstv.at[b], eav.at[b], ev.at[b], rows.at[b]
        pltpu.make_async_copy(h_hbm.at[sv], rb, gsem.at[b]).wait()
        pltpu.make_async_copy(sd_hbm.at[dv], sdg.at[b], gsem.at[b]).wait()
        pltpu.make_async_copy(ss_hbm.at[sv], ssg.at[b], gsem.at[b]).wait()

        def group_body(g, carry):
            sl = pl.ds(g * 16, 16)
            si = sv[sl]
            di = dv[sl]
            alpha = sdg.at[b][sl] + ssg.at[b][sl] + av[sl]
            alpha = jnp.where(alpha >= 0.0, alpha, 0.2 * alpha)
            gidx = base + g * 16 + lax.iota(_i32, 16)
            keep = (si != di) | (gidx >= E)
            valid = gidx < EPRIME
            mf = jnp.where(keep & valid, 1.0, 0.0).astype(_f32)
            e16 = mf * jnp.exp(alpha)
            evb[sl] = e16
            for r in range(16):
                eb = _lane_bcast(e16, r)
                gr = g * 16 + r
                for cc in range(D // 16):
                    csl = pl.ds(cc * 16, 16)
                    rb[gr, csl] = rb[gr, csl] * eb
            return carry

        lax.fori_loop(0, CH // 16, group_body, 0)
        pltpu.async_copy(rb, aggr_sh.at[dv], asem.at[b], add=True)
        pltpu.async_copy(evb, den_sh.at[dv], asem.at[b], add=True)

    def drain(b):
        pltpu.make_async_copy(rows.at[b], aggr_sh.at[dstv.at[b]],
                              asem.at[b]).wait()
        pltpu.make_async_copy(ev.at[b], den_sh.at[dstv.at[b]],
                              asem.at[b]).wait()

    fetch(0, 0)
    fetch(1, 1)

    def outer_body(i, carry):
        for j in range(NBUF):
            c = i * NBUF + j
            bf = (j + 2) % NBUF

            @pl.when((c >= 2) & (c + 2 < NCHUNK))
            def _():
                drain(bf)

            @pl.when(c + 2 < NCHUNK)
            def _():
                fetch(c + 2, bf)

            compute_and_scatter(c, j)
        return carry

    lax.fori_loop(0, NCHUNK // NBUF, outer_body, 0)
    for b in range(NBUF):
        drain(b)

    plsc.subcore_barrier()
    sl_rows = pl.ds(sid * ROWS_PER_TILE, ROWS_PER_TILE)
    pltpu.sync_copy(aggr_sh.at[sl_rows], aggr_out.at[cid, sl_rows])
    pltpu.sync_copy(den_sh.at[sl_rows],
                    den_out.at[pl.ds(cid * NPAD + sid * ROWS_PER_TILE,
                                     ROWS_PER_TILE)])


def _sc_edge(src, dst, eaw, sd, ss, h, zrow, zvec):
    mesh = plsc.VectorSubcoreMesh(core_axis_name="c", subcore_axis_name="s",
                                  num_cores=2, num_subcores=16)
    fn = pl.kernel(
        _sc_edge_body,
        out_type=(
            jax.ShapeDtypeStruct((2, NPAD, D), _f32),
            jax.ShapeDtypeStruct((2 * NPAD,), _f32),
        ),
        mesh=mesh,
        compiler_params=pltpu.CompilerParams(needs_layout_passes=False),
        scratch_types=[
            pltpu.VMEM((NBUF, CH), _i32),
            pltpu.VMEM((NBUF, CH), _i32),
            pltpu.VMEM((NBUF, CH), _f32),
            pltpu.VMEM((NBUF, CH), _f32),
            pltpu.VMEM((NBUF, CH), _f32),
            pltpu.VMEM((NBUF, CH), _f32),
            pltpu.VMEM((NBUF, CH, D), _f32),
            pltpu.VMEM_SHARED((NPAD, D), _f32),
            pltpu.VMEM_SHARED((NPAD,), _f32),
            pltpu.SemaphoreType.DMA((NBUF,)),
            pltpu.SemaphoreType.DMA((NBUF,)),
        ],
    )
    return fn(src, dst, eaw, sd, ss, h, zrow, zvec)


def kernel(x, edge_index, edge_attr, batch, W0, att0, b0, W1, att1, b1, Wf, bf):
    loop = jnp.arange(N, dtype=_i32)
    pad_e = jnp.zeros((EARR - EPRIME,), _i32)
    src = jnp.concatenate([edge_index[0], loop, pad_e])
    dst = jnp.concatenate([edge_index[1], loop, pad_e])
    ea = jnp.concatenate([edge_attr, jnp.zeros((N + EARR - EPRIME,), _f32)])
    ea2d = ea.reshape(EARR // 128, 128)

    x_pad = jnp.pad(x, ((0, NPAD - N), (0, 0)))
    batchcol = jnp.concatenate(
        [batch.astype(_i32), jnp.full((NPAD - N,), NB, _i32)]).reshape(NPAD, 1)

    zrow = jnp.zeros((ROWS_PER_TILE, D), _f32)
    zvec = jnp.zeros((ROWS_PER_TILE,), _f32)

    h0, sd0, ss0, eaw0 = _tc_feats(x_pad, W0, att0, ea2d)
    aggr0, den0 = _sc_edge(src, dst, eaw0.reshape(EARR), sd0.reshape(NPAD),
                           ss0.reshape(NPAD), h0, zrow, zvec)

    h1, sd1, ss1, eaw1 = _tc_combine(aggr0, den0.reshape(2, NPAD, 1), b0,
                                     W1, att1, ea2d)
    aggr1, den1 = _sc_edge(src, dst, eaw1.reshape(EARR), sd1.reshape(NPAD),
                           ss1.reshape(NPAD), h1, zrow, zvec)

    y = _tc_final(aggr1, den1.reshape(2, NPAD, 1), b1, batchcol, Wf, bf)
    return y.reshape(NB)


# speedup vs baseline: 1.2296x; 1.2296x over previous
import functools

import jax
import jax.numpy as jnp
from jax import lax
from jax.experimental import pallas as pl
from jax.experimental.pallas import tpu as pltpu
from jax.experimental.pallas import tpu_sc as plsc

N = 10000
E = 320000
EPRIME = E + N
D = 128
H = 128
NB = 64

NPAD = 10240
CH = 64
NTILES = 32
NBUF = 3
NCHUNK = 168
EARR = NTILES * CH * NCHUNK
ROWS_PER_TILE = NPAD // 16

_f32 = jnp.float32
_i32 = jnp.int32


def _tc_feats_body(x_ref, w_ref, att_ref, ea_ref, h_ref, sd_ref, ss_ref, eaw_ref):
    h = jnp.dot(x_ref[...], w_ref[...], preferred_element_type=_f32)
    h_ref[...] = h
    att = att_ref[0, 0, :]
    att_d = att[0:H].reshape(H, 1)
    att_s = att[H:2 * H].reshape(H, 1)
    sd_ref[...] = jnp.dot(h, att_d, preferred_element_type=_f32)
    ss_ref[...] = jnp.dot(h, att_s, preferred_element_type=_f32)
    eaw_ref[...] = ea_ref[...] * att_ref[0, 0, 2 * H]


def _tc_feats(x_pad, w, att, ea2d):
    return pl.pallas_call(
        _tc_feats_body,
        out_shape=[
            jax.ShapeDtypeStruct((NPAD, D), _f32),
            jax.ShapeDtypeStruct((NPAD, 1), _f32),
            jax.ShapeDtypeStruct((NPAD, 1), _f32),
            jax.ShapeDtypeStruct((EARR // 128, 128), _f32),
        ],
    )(x_pad, w, att, ea2d)


def _tc_combine_body(ag_ref, den_ref, b_ref, w_ref, att_ref, ea_ref,
                     h_ref, sd_ref, ss_ref, eaw_ref):
    a = ag_ref[0] + ag_ref[1]
    dsum = den_ref[0] + den_ref[1]
    hin = jnp.maximum(a / (dsum + 1e-16) + b_ref[...][None, :], 0.0)
    h = jnp.dot(hin, w_ref[...], preferred_element_type=_f32)
    h_ref[...] = h
    att = att_ref[0, 0, :]
    att_d = att[0:H].reshape(H, 1)
    att_s = att[H:2 * H].reshape(H, 1)
    sd_ref[...] = jnp.dot(h, att_d, preferred_element_type=_f32)
    ss_ref[...] = jnp.dot(h, att_s, preferred_element_type=_f32)
    eaw_ref[...] = ea_ref[...] * att_ref[0, 0, 2 * H]


def _tc_combine(aggr, den3, b, w, att, ea2d):
    return pl.pallas_call(
        _tc_combine_body,
        out_shape=[
            jax.ShapeDtypeStruct((NPAD, D), _f32),
            jax.ShapeDtypeStruct((NPAD, 1), _f32),
            jax.ShapeDtypeStruct((NPAD, 1), _f32),
            jax.ShapeDtypeStruct((EARR // 128, 128), _f32),
        ],
    )(aggr, den3, b, w, att, ea2d)


def _tc_final_body(ag_ref, den_ref, b_ref, batch_ref, wf_ref, bf_ref, y_ref):
    a = ag_ref[0] + ag_ref[1]
    dsum = den_ref[0] + den_ref[1]
    h = jnp.maximum(a / (dsum + 1e-16) + b_ref[...][None, :], 0.0)
    ids = lax.broadcasted_iota(_i32, (1, NB), 1)
    oh = (batch_ref[...] == ids).astype(_f32)
    pooled = lax.dot_general(oh, h, (((0,), (0,)), ((), ())),
                             preferred_element_type=_f32)
    y_ref[...] = jnp.dot(pooled, wf_ref[...], preferred_element_type=_f32) + bf_ref[0]


def _tc_final(aggr, den3, b, batchcol, wf, bf):
    return pl.pallas_call(
        _tc_final_body,
        out_shape=jax.ShapeDtypeStruct((NB, 1), _f32),
    )(aggr, den3, b, batchcol, wf, bf)


def _lane_bcast(v, lane):
    idx = jnp.full((16, 1), lane, _i32)
    dnums = lax.GatherDimensionNumbers(offset_dims=(), collapsed_slice_dims=(0,),
                                       start_index_map=(0,))
    return lax.gather(v, idx, dnums, (1,),
                      mode=lax.GatherScatterMode.PROMISE_IN_BOUNDS)


def _sc_edge_body(src_hbm, dst_hbm, eaw_hbm, sd_hbm, ss_hbm, h_hbm,
                  zrow_hbm, zvec_hbm,
                  aggr_out, den_out,
                  sd_v, ss_v, srcv, dstv, eav, ev, rows,
                  aggr_sh, den_sh, gsem, asem):
    cid = lax.axis_index("c")
    sid = lax.axis_index("s")
    wid = cid * 16 + sid

    pltpu.sync_copy(zrow_hbm, aggr_sh.at[pl.ds(sid * ROWS_PER_TILE, ROWS_PER_TILE)])
    pltpu.sync_copy(zvec_hbm, den_sh.at[pl.ds(sid * ROWS_PER_TILE, ROWS_PER_TILE)])

    pltpu.sync_copy(sd_hbm, sd_v)
    pltpu.sync_copy(ss_hbm, ss_v)
    plsc.subcore_barrier()

    def fetch(c, b):
        base = (wid * NCHUNK + c) * CH
        pltpu.sync_copy(src_hbm.at[pl.ds(base, CH)], srcv.at[b])
        pltpu.sync_copy(dst_hbm.at[pl.ds(base, CH)], dstv.at[b])
        pltpu.sync_copy(eaw_hbm.at[pl.ds(base, CH)], eav.at[b])
        pltpu.async_copy(h_hbm.at[srcv.at[b]], rows.at[b], gsem.at[b])

    def compute_and_scatter(c, b):
        base = (wid * NCHUNK + c) * CH
        sv, dv, av, evb, rb = srcv.at[b], dstv.at[b], eav.at[b], ev.at[b], rows.at[b]
        pltpu.make_async_copy(h_hbm.at[sv], rb, gsem.at[b]).wait()

        def group_body(g, carry):
            sl = pl.ds(g * 16, 16)
            si = sv[sl]
            di = dv[sl]
            svals = plsc.load_gather(ss_v, [si])
            dvals = plsc.load_gather(sd_v, [di])
            alpha = svals + dvals + av[sl]
            alpha = jnp.where(alpha >= 0.0, alpha, 0.2 * alpha)
            gidx = base + g * 16 + lax.iota(_i32, 16)
            keep = (si != di) | (gidx >= E)
            valid = gidx < EPRIME
            mf = jnp.where(keep & valid, 1.0, 0.0).astype(_f32)
            e16 = mf * jnp.exp(alpha)
            evb[sl] = e16
            for r in range(16):
                eb = _lane_bcast(e16, r)
                gr = g * 16 + r
                for cc in range(D // 16):
                    csl = pl.ds(cc * 16, 16)
                    rb[gr, csl] = rb[gr, csl] * eb
            return carry

        lax.fori_loop(0, CH // 16, group_body, 0)
        pltpu.async_copy(rb, aggr_sh.at[dv], asem.at[b], add=True)
        pltpu.async_copy(evb, den_sh.at[dv], asem.at[b], add=True)

    def drain(b):
        pltpu.make_async_copy(rows.at[b], aggr_sh.at[dstv.at[b]],
                              asem.at[b]).wait()
        pltpu.make_async_copy(ev.at[b], den_sh.at[dstv.at[b]],
                              asem.at[b]).wait()

    fetch(0, 0)

    def outer_body(i, carry):
        for j in range(NBUF):
            c = i * NBUF + j
            bf = (j + 1) % NBUF

            @pl.when((c >= 2) & (c + 1 < NCHUNK))
            def _():
                drain(bf)

            @pl.when(c + 1 < NCHUNK)
            def _():
                fetch(c + 1, bf)

            compute_and_scatter(c, j)
        return carry

    lax.fori_loop(0, NCHUNK // NBUF, outer_body, 0)
    for b in range(NBUF):
        drain(b)

    plsc.subcore_barrier()
    sl_rows = pl.ds(sid * ROWS_PER_TILE, ROWS_PER_TILE)
    pltpu.sync_copy(aggr_sh.at[sl_rows], aggr_out.at[cid, sl_rows])
    pltpu.sync_copy(den_sh.at[sl_rows],
                    den_out.at[pl.ds(cid * NPAD + sid * ROWS_PER_TILE,
                                     ROWS_PER_TILE)])


def _sc_edge(src, dst, eaw, sd, ss, h, zrow, zvec):
    mesh = plsc.VectorSubcoreMesh(core_axis_name="c", subcore_axis_name="s",
                                  num_cores=2, num_subcores=16)
    fn = pl.kernel(
        _sc_edge_body,
        out_type=(
            jax.ShapeDtypeStruct((2, NPAD, D), _f32),
            jax.ShapeDtypeStruct((2 * NPAD,), _f32),
        ),
        mesh=mesh,
        compiler_params=pltpu.CompilerParams(needs_layout_passes=False),
        scratch_types=[
            pltpu.VMEM((NPAD,), _f32),
            pltpu.VMEM((NPAD,), _f32),
            pltpu.VMEM((NBUF, CH), _i32),
            pltpu.VMEM((NBUF, CH), _i32),
            pltpu.VMEM((NBUF, CH), _f32),
            pltpu.VMEM((NBUF, CH), _f32),
            pltpu.VMEM((NBUF, CH, D), _f32),
            pltpu.VMEM_SHARED((NPAD, D), _f32),
            pltpu.VMEM_SHARED((NPAD,), _f32),
            pltpu.SemaphoreType.DMA((NBUF,)),
            pltpu.SemaphoreType.DMA((NBUF,)),
        ],
    )
    return fn(src, dst, eaw, sd, ss, h, zrow, zvec)


def kernel(x, edge_index, edge_attr, batch, W0, att0, b0, W1, att1, b1, Wf, bf):
    loop = jnp.arange(N, dtype=_i32)
    pad_e = jnp.zeros((EARR - EPRIME,), _i32)
    src = jnp.concatenate([edge_index[0], loop, pad_e])
    dst = jnp.concatenate([edge_index[1], loop, pad_e])
    ea = jnp.concatenate([edge_attr, jnp.zeros((N + EARR - EPRIME,), _f32)])
    ea2d = ea.reshape(EARR // 128, 128)

    x_pad = jnp.pad(x, ((0, NPAD - N), (0, 0)))
    batchcol = jnp.concatenate(
        [batch.astype(_i32), jnp.full((NPAD - N,), NB, _i32)]).reshape(NPAD, 1)

    zrow = jnp.zeros((ROWS_PER_TILE, D), _f32)
    zvec = jnp.zeros((ROWS_PER_TILE,), _f32)

    h0, sd0, ss0, eaw0 = _tc_feats(x_pad, W0, att0, ea2d)
    aggr0, den0 = _sc_edge(src, dst, eaw0.reshape(EARR), sd0.reshape(NPAD),
                           ss0.reshape(NPAD), h0, zrow, zvec)

    h1, sd1, ss1, eaw1 = _tc_combine(aggr0, den0.reshape(2, NPAD, 1), b0,
                                     W1, att1, ea2d)
    aggr1, den1 = _sc_edge(src, dst, eaw1.reshape(EARR), sd1.reshape(NPAD),
                           ss1.reshape(NPAD), h1, zrow, zvec)

    y = _tc_final(aggr1, den1.reshape(2, NPAD, 1), b1, batchcol, Wf, bf)
    return y.reshape(NB)


# speedup vs baseline: 2.3430x; 1.9055x over previous
import jax
import jax.numpy as jnp
from jax import lax
from jax.experimental import pallas as pl
from jax.experimental.pallas import tpu as pltpu
from jax.experimental.pallas import tpu_sc as plsc

N = 10000
E = 320000
EPRIME = E + N
D = 128
H = 128
NB = 64

NPAD = 10240
CH = 128
NTILES = 32
NCHUNK = 81
EARR = NTILES * CH * NCHUNK
ROWS_PER_TILE = NPAD // 16
NGRP = CH // 16

_f32 = jnp.float32
_i32 = jnp.int32


def _proj(h, att_ref):
    att = att_ref[0, 0, :]
    att_d = att[0:H].reshape(H, 1)
    att_s = att[H:2 * H].reshape(H, 1)
    sd = jnp.dot(h, att_d, preferred_element_type=_f32)
    ss = jnp.dot(h, att_s, preferred_element_type=_f32)
    return sd, ss


def _tc_feats_body(x_ref, w_ref, att_ref, h_ref, sd_ref, ss_ref):
    h = jnp.dot(x_ref[...], w_ref[...], preferred_element_type=_f32)
    h_ref[...] = h
    sd_ref[...], ss_ref[...] = _proj(h, att_ref)


def _tc_feats(x_pad, w, att):
    return pl.pallas_call(
        _tc_feats_body,
        out_shape=[
            jax.ShapeDtypeStruct((NPAD, D), _f32),
            jax.ShapeDtypeStruct((NPAD, 1), _f32),
            jax.ShapeDtypeStruct((NPAD, 1), _f32),
        ],
    )(x_pad, w, att)


def _tc_combine_body(ag_ref, den_ref, b_ref, w_ref, att_ref,
                     h_ref, sd_ref, ss_ref):
    a = ag_ref[0] + ag_ref[1]
    dsum = den_ref[0] + den_ref[1]
    hin = jnp.maximum(a / (dsum + 1e-16) + b_ref[...][None, :], 0.0)
    h = jnp.dot(hin, w_ref[...], preferred_element_type=_f32)
    h_ref[...] = h
    sd_ref[...], ss_ref[...] = _proj(h, att_ref)


def _tc_combine(aggr, den3, b, w, att):
    return pl.pallas_call(
        _tc_combine_body,
        out_shape=[
            jax.ShapeDtypeStruct((NPAD, D), _f32),
            jax.ShapeDtypeStruct((NPAD, 1), _f32),
            jax.ShapeDtypeStruct((NPAD, 1), _f32),
        ],
    )(aggr, den3, b, w, att)


def _tc_final_body(ag_ref, den_ref, b_ref, batch_ref, wf_ref, bf_ref, y_ref):
    a = ag_ref[0] + ag_ref[1]
    dsum = den_ref[0] + den_ref[1]
    h = jnp.maximum(a / (dsum + 1e-16) + b_ref[...][None, :], 0.0)
    ids = lax.broadcasted_iota(_i32, (1, NB), 1)
    oh = (batch_ref[...] == ids).astype(_f32)
    pooled = lax.dot_general(oh, h, (((0,), (0,)), ((), ())),
                             preferred_element_type=_f32)
    y_ref[...] = jnp.dot(pooled, wf_ref[...], preferred_element_type=_f32) + bf_ref[0]


def _tc_final(aggr, den3, b, batchcol, wf, bf):
    return pl.pallas_call(
        _tc_final_body,
        out_shape=jax.ShapeDtypeStruct((NB, 1), _f32),
    )(aggr, den3, b, batchcol, wf, bf)


def _lane_bcast(v, lane):
    idx = jnp.full((16, 1), lane, _i32)
    dnums = lax.GatherDimensionNumbers(offset_dims=(), collapsed_slice_dims=(0,),
                                       start_index_map=(0,))
    return lax.gather(v, idx, dnums, (1,),
                      mode=lax.GatherScatterMode.PROMISE_IN_BOUNDS)


def _sc_edge_body(edata_hbm, attv_hbm, sd_hbm, ss_hbm, h_hbm,
                  zrow_hbm, zvec_hbm,
                  aggr_out, den_out,
                  sd_v, ss_v, attv_v, edv, ev, rows,
                  aggr_sh, den_sh):
    cid = lax.axis_index("c")
    sid = lax.axis_index("s")
    wid = cid * 16 + sid

    pltpu.sync_copy(zrow_hbm, aggr_sh.at[pl.ds(sid * ROWS_PER_TILE, ROWS_PER_TILE)])
    pltpu.sync_copy(zvec_hbm, den_sh.at[pl.ds(sid * ROWS_PER_TILE, ROWS_PER_TILE)])

    pltpu.sync_copy(sd_hbm, sd_v)
    pltpu.sync_copy(ss_hbm, ss_v)
    pltpu.sync_copy(attv_hbm, attv_v)
    plsc.subcore_barrier()

    def chunk_body(c, carry):
        ci = wid * NCHUNK + c
        base = ci * CH
        pltpu.sync_copy(edata_hbm.at[ci], edv)
        sv = edv.at[0]
        dv = edv.at[1]
        pltpu.sync_copy(h_hbm.at[sv], rows)
        atte = attv_v[pl.ds(0, 16)]

        def group_body(g, carry2):
            sl = pl.ds(g * 16, 16)
            si = edv[0, sl]
            di = edv[1, sl]
            eab = plsc.bitcast(edv[2, sl], _f32)
            svals = plsc.load_gather(ss_v, [si])
            dvals = plsc.load_gather(sd_v, [di])
            alpha = svals + dvals + eab * atte
            alpha = jnp.where(alpha >= 0.0, alpha, 0.2 * alpha)
            gidx = base + g * 16 + lax.iota(_i32, 16)
            keep = (si != di) | (gidx >= E)
            valid = gidx < EPRIME
            mf = jnp.where(keep & valid, 1.0, 0.0).astype(_f32)
            e16 = mf * jnp.exp(alpha)
            ev[sl] = e16
            for r in range(16):
                eb = _lane_bcast(e16, r)
                gr = g * 16 + r
                for cc in range(D // 16):
                    csl = pl.ds(cc * 16, 16)
                    rows[gr, csl] = rows[gr, csl] * eb
            return carry2

        lax.fori_loop(0, NGRP, group_body, 0)
        pltpu.sync_copy(rows, aggr_sh.at[dv], add=True)
        pltpu.sync_copy(ev, den_sh.at[dv], add=True)
        return carry

    lax.fori_loop(0, NCHUNK, chunk_body, 0)

    plsc.subcore_barrier()
    sl_rows = pl.ds(sid * ROWS_PER_TILE, ROWS_PER_TILE)
    pltpu.sync_copy(aggr_sh.at[sl_rows], aggr_out.at[cid, sl_rows])
    pltpu.sync_copy(den_sh.at[sl_rows],
                    den_out.at[pl.ds(cid * NPAD + sid * ROWS_PER_TILE,
                                     ROWS_PER_TILE)])


def _sc_edge(edata, attv, sd, ss, h, zrow, zvec):
    mesh = plsc.VectorSubcoreMesh(core_axis_name="c", subcore_axis_name="s",
                                  num_cores=2, num_subcores=16)
    fn = pl.kernel(
        _sc_edge_body,
        out_type=(
            jax.ShapeDtypeStruct((2, NPAD, D), _f32),
            jax.ShapeDtypeStruct((2 * NPAD,), _f32),
        ),
        mesh=mesh,
        compiler_params=pltpu.CompilerParams(needs_layout_passes=False),
        scratch_types=[
            pltpu.VMEM((NPAD,), _f32),
            pltpu.VMEM((NPAD,), _f32),
            pltpu.VMEM((16,), _f32),
            pltpu.VMEM((3, CH), _i32),
            pltpu.VMEM((CH,), _f32),
            pltpu.VMEM((CH, D), _f32),
            pltpu.VMEM_SHARED((NPAD, D), _f32),
            pltpu.VMEM_SHARED((NPAD,), _f32),
        ],
    )
    return fn(edata, attv, sd, ss, h, zrow, zvec)


def kernel(x, edge_index, edge_attr, batch, W0, att0, b0, W1, att1, b1, Wf, bf):
    loop = jnp.arange(N, dtype=_i32)
    pad_e = jnp.zeros((EARR - EPRIME,), _i32)
    src = jnp.concatenate([edge_index[0], loop, pad_e])
    dst = jnp.concatenate([edge_index[1], loop, pad_e])
    ea = jnp.concatenate([edge_attr, jnp.zeros((N + EARR - EPRIME,), _f32)])
    eabits = lax.bitcast_convert_type(ea, _i32)

    nchunks_total = EARR // CH
    edata = jnp.stack([src.reshape(nchunks_total, CH),
                       dst.reshape(nchunks_total, CH),
                       eabits.reshape(nchunks_total, CH)], axis=1)

    attv0 = jnp.broadcast_to(att0[0, 0, 2 * H:2 * H + 1], (16,)).astype(_f32)
    attv1 = jnp.broadcast_to(att1[0, 0, 2 * H:2 * H + 1], (16,)).astype(_f32)

    x_pad = jnp.pad(x, ((0, NPAD - N), (0, 0)))
    batchcol = jnp.concatenate(
        [batch.astype(_i32), jnp.full((NPAD - N,), NB, _i32)]).reshape(NPAD, 1)

    zrow = jnp.zeros((ROWS_PER_TILE, D), _f32)
    zvec = jnp.zeros((ROWS_PER_TILE,), _f32)

    h0, sd0, ss0 = _tc_feats(x_pad, W0, att0)
    aggr0, den0 = _sc_edge(edata, attv0, sd0.reshape(NPAD), ss0.reshape(NPAD),
                           h0, zrow, zvec)

    h1, sd1, ss1 = _tc_combine(aggr0, den0.reshape(2, NPAD, 1), b0, W1, att1)
    aggr1, den1 = _sc_edge(edata, attv1, sd1.reshape(NPAD), ss1.reshape(NPAD),
                           h1, zrow, zvec)

    y = _tc_final(aggr1, den1.reshape(2, NPAD, 1), b1, batchcol, Wf, bf)
    return y.reshape(NB)


# speedup vs baseline: 2.4992x; 1.0667x over previous
import jax
import jax.numpy as jnp
from jax import lax
from jax.experimental import pallas as pl
from jax.experimental.pallas import tpu as pltpu
from jax.experimental.pallas import tpu_sc as plsc

N = 10000
E = 320000
EPRIME = E + N
D = 128
H = 128
NB = 64

NPAD = 10240
CH = 128
NTILES = 32
NCHUNK = 81
EBLK = 3
EARR = NTILES * CH * NCHUNK
ROWS_PER_TILE = NPAD // 16
NGRP = CH // 16

_f32 = jnp.float32
_i32 = jnp.int32


def _proj(h, att_ref):
    att = att_ref[0, 0, :]
    att_d = att[0:H].reshape(H, 1)
    att_s = att[H:2 * H].reshape(H, 1)
    sd = jnp.dot(h, att_d, preferred_element_type=_f32)
    ss = jnp.dot(h, att_s, preferred_element_type=_f32)
    return sd, ss


def _tc_feats_body(x_ref, w_ref, att_ref, h_ref, sd_ref, ss_ref):
    h = jnp.dot(x_ref[...], w_ref[...], preferred_element_type=_f32)
    h_ref[...] = h
    sd_ref[...], ss_ref[...] = _proj(h, att_ref)


def _tc_feats(x_pad, w, att):
    return pl.pallas_call(
        _tc_feats_body,
        out_shape=[
            jax.ShapeDtypeStruct((NPAD, D), _f32),
            jax.ShapeDtypeStruct((NPAD, 1), _f32),
            jax.ShapeDtypeStruct((NPAD, 1), _f32),
        ],
    )(x_pad, w, att)


def _tc_combine_body(ag_ref, den_ref, b_ref, w_ref, att_ref,
                     h_ref, sd_ref, ss_ref):
    a = ag_ref[0] + ag_ref[1]
    dsum = den_ref[0] + den_ref[1]
    hin = jnp.maximum(a / (dsum + 1e-16) + b_ref[...][None, :], 0.0)
    h = jnp.dot(hin, w_ref[...], preferred_element_type=_f32)
    h_ref[...] = h
    sd_ref[...], ss_ref[...] = _proj(h, att_ref)


def _tc_combine(aggr, den3, b, w, att):
    return pl.pallas_call(
        _tc_combine_body,
        out_shape=[
            jax.ShapeDtypeStruct((NPAD, D), _f32),
            jax.ShapeDtypeStruct((NPAD, 1), _f32),
            jax.ShapeDtypeStruct((NPAD, 1), _f32),
        ],
    )(aggr, den3, b, w, att)


def _tc_final_body(ag_ref, den_ref, b_ref, batch_ref, wf_ref, bf_ref, y_ref):
    a = ag_ref[0] + ag_ref[1]
    dsum = den_ref[0] + den_ref[1]
    h = jnp.maximum(a / (dsum + 1e-16) + b_ref[...][None, :], 0.0)
    ids = lax.broadcasted_iota(_i32, (1, NB), 1)
    oh = (batch_ref[...] == ids).astype(_f32)
    pooled = lax.dot_general(oh, h, (((0,), (0,)), ((), ())),
                             preferred_element_type=_f32)
    y_ref[...] = jnp.dot(pooled, wf_ref[...], preferred_element_type=_f32) + bf_ref[0]


def _tc_final(aggr, den3, b, batchcol, wf, bf):
    return pl.pallas_call(
        _tc_final_body,
        out_shape=jax.ShapeDtypeStruct((NB, 1), _f32),
    )(aggr, den3, b, batchcol, wf, bf)


def _lane_bcast(v, lane):
    idx = jnp.full((16, 1), lane, _i32)
    dnums = lax.GatherDimensionNumbers(offset_dims=(), collapsed_slice_dims=(0,),
                                       start_index_map=(0,))
    return lax.gather(v, idx, dnums, (1,),
                      mode=lax.GatherScatterMode.PROMISE_IN_BOUNDS)


def _sc_edge_body(edata_hbm, attv_hbm, sd_hbm, ss_hbm, h_hbm,
                  zrow_hbm, zvec_hbm,
                  aggr_out, den_out,
                  sd_v, ss_v, attv_v, edv, ev, rows,
                  aggr_sh, den_sh):
    cid = lax.axis_index("c")
    sid = lax.axis_index("s")
    wid = cid * 16 + sid

    pltpu.sync_copy(zrow_hbm, aggr_sh.at[pl.ds(sid * ROWS_PER_TILE, ROWS_PER_TILE)])
    pltpu.sync_copy(zvec_hbm, den_sh.at[pl.ds(sid * ROWS_PER_TILE, ROWS_PER_TILE)])

    pltpu.sync_copy(sd_hbm, sd_v)
    pltpu.sync_copy(ss_hbm, ss_v)
    pltpu.sync_copy(attv_hbm, attv_v)
    plsc.subcore_barrier()

    def block_body(blk, carry):
        pltpu.sync_copy(edata_hbm.at[wid * (NCHUNK // EBLK) + blk], edv)
        atte = attv_v[pl.ds(0, 16)]
        for j in range(EBLK):
            c = blk * EBLK + j
            base = (wid * NCHUNK + c) * CH
            dv = edv.at[j * 3 + 1]
            pltpu.sync_copy(h_hbm.at[edv.at[j * 3]], rows)

            def group_body(g, carry2, _j=j, _base=base):
                sl = pl.ds(g * 16, 16)
                si = edv[_j * 3, sl]
                di = edv[_j * 3 + 1, sl]
                eab = plsc.bitcast(edv[_j * 3 + 2, sl], _f32)
                svals = plsc.load_gather(ss_v, [si])
                dvals = plsc.load_gather(sd_v, [di])
                alpha = svals + dvals + eab * atte
                alpha = jnp.where(alpha >= 0.0, alpha, 0.2 * alpha)
                gidx = _base + g * 16 + lax.iota(_i32, 16)
                keep = (si != di) | (gidx >= E)
                valid = gidx < EPRIME
                mf = jnp.where(keep & valid, 1.0, 0.0).astype(_f32)
                e16 = mf * jnp.exp(alpha)
                ev[sl] = e16
                for r in range(16):
                    eb = _lane_bcast(e16, r)
                    gr = g * 16 + r
                    for cc in range(D // 16):
                        csl = pl.ds(cc * 16, 16)
                        rows[gr, csl] = rows[gr, csl] * eb
                return carry2

            lax.fori_loop(0, NGRP, group_body, 0)
            pltpu.sync_copy(rows, aggr_sh.at[dv], add=True)
            pltpu.sync_copy(ev, den_sh.at[dv], add=True)
        return carry

    lax.fori_loop(0, NCHUNK // EBLK, block_body, 0)

    plsc.subcore_barrier()
    sl_rows = pl.ds(sid * ROWS_PER_TILE, ROWS_PER_TILE)
    pltpu.sync_copy(aggr_sh.at[sl_rows], aggr_out.at[cid, sl_rows])
    pltpu.sync_copy(den_sh.at[sl_rows],
                    den_out.at[pl.ds(cid * NPAD + sid * ROWS_PER_TILE,
                                     ROWS_PER_TILE)])


def _sc_edge(edata, attv, sd, ss, h, zrow, zvec):
    mesh = plsc.VectorSubcoreMesh(core_axis_name="c", subcore_axis_name="s",
                                  num_cores=2, num_subcores=16)
    fn = pl.kernel(
        _sc_edge_body,
        out_type=(
            jax.ShapeDtypeStruct((2, NPAD, D), _f32),
            jax.ShapeDtypeStruct((2 * NPAD,), _f32),
        ),
        mesh=mesh,
        compiler_params=pltpu.CompilerParams(needs_layout_passes=False),
        scratch_types=[
            pltpu.VMEM((NPAD,), _f32),
            pltpu.VMEM((NPAD,), _f32),
            pltpu.VMEM((16,), _f32),
            pltpu.VMEM((EBLK * 3, CH), _i32),
            pltpu.VMEM((CH,), _f32),
            pltpu.VMEM((CH, D), _f32),
            pltpu.VMEM_SHARED((NPAD, D), _f32),
            pltpu.VMEM_SHARED((NPAD,), _f32),
        ],
    )
    return fn(edata, attv, sd, ss, h, zrow, zvec)


def kernel(x, edge_index, edge_attr, batch, W0, att0, b0, W1, att1, b1, Wf, bf):
    loop = jnp.arange(N, dtype=_i32)
    pad_e = jnp.zeros((EARR - EPRIME,), _i32)
    src = jnp.concatenate([edge_index[0], loop, pad_e])
    dst = jnp.concatenate([edge_index[1], loop, pad_e])
    ea = jnp.concatenate([edge_attr, jnp.zeros((N + EARR - EPRIME,), _f32)])
    eabits = lax.bitcast_convert_type(ea, _i32)

    nchunks_total = EARR // CH
    edata = jnp.stack([src.reshape(nchunks_total, CH),
                       dst.reshape(nchunks_total, CH),
                       eabits.reshape(nchunks_total, CH)],
                      axis=1).reshape(nchunks_total // EBLK, EBLK * 3, CH)

    attv0 = jnp.broadcast_to(att0[0, 0, 2 * H:2 * H + 1], (16,)).astype(_f32)
    attv1 = jnp.broadcast_to(att1[0, 0, 2 * H:2 * H + 1], (16,)).astype(_f32)

    x_pad = jnp.pad(x, ((0, NPAD - N), (0, 0)))
    batchcol = jnp.concatenate(
        [batch.astype(_i32), jnp.full((NPAD - N,), NB, _i32)]).reshape(NPAD, 1)

    zrow = jnp.zeros((ROWS_PER_TILE, D), _f32)
    zvec = jnp.zeros((ROWS_PER_TILE,), _f32)

    h0, sd0, ss0 = _tc_feats(x_pad, W0, att0)
    aggr0, den0 = _sc_edge(edata, attv0, sd0.reshape(NPAD), ss0.reshape(NPAD),
                           h0, zrow, zvec)

    h1, sd1, ss1 = _tc_combine(aggr0, den0.reshape(2, NPAD, 1), b0, W1, att1)
    aggr1, den1 = _sc_edge(edata, attv1, sd1.reshape(NPAD), ss1.reshape(NPAD),
                           h1, zrow, zvec)

    y = _tc_final(aggr1, den1.reshape(2, NPAD, 1), b1, batchcol, Wf, bf)
    return y.reshape(NB)


# speedup vs baseline: 2.5859x; 1.0347x over previous
import jax
import jax.numpy as jnp
from jax import lax
from jax.experimental import pallas as pl
from jax.experimental.pallas import tpu as pltpu
from jax.experimental.pallas import tpu_sc as plsc

N = 10000
E = 320000
EPRIME = E + N
D = 128
H = 128
NB = 64

NPAD = 10240
CH = 128
NTILES = 32
NCHUNK = 81
EBLK = 3
EARR = NTILES * CH * NCHUNK
ROWS_PER_TILE = NPAD // 16
NGRP = CH // 16

_f32 = jnp.float32
_i32 = jnp.int32


def _proj(h, att_ref):
    att = att_ref[0, 0, :]
    att_d = att[0:H].reshape(H, 1)
    att_s = att[H:2 * H].reshape(H, 1)
    sd = jnp.dot(h, att_d, preferred_element_type=_f32)
    ss = jnp.dot(h, att_s, preferred_element_type=_f32)
    return sd, ss


def _tc_feats_body(x_ref, w_ref, att_ref, h_ref, sd_ref, ss_ref):
    h = jnp.dot(x_ref[...], w_ref[...], preferred_element_type=_f32)
    h_ref[...] = h
    sd_ref[...], ss_ref[...] = _proj(h, att_ref)


def _tc_feats(x_pad, w, att):
    return pl.pallas_call(
        _tc_feats_body,
        out_shape=[
            jax.ShapeDtypeStruct((NPAD, D), _f32),
            jax.ShapeDtypeStruct((NPAD, 1), _f32),
            jax.ShapeDtypeStruct((NPAD, 1), _f32),
        ],
    )(x_pad, w, att)


def _tc_combine_body(ag_ref, den_ref, b_ref, w_ref, att_ref,
                     h_ref, sd_ref, ss_ref):
    a = ag_ref[0] + ag_ref[1]
    dsum = den_ref[...]
    hin = jnp.maximum(a / (dsum + 1e-16) + b_ref[...][None, :], 0.0)
    h = jnp.dot(hin, w_ref[...], preferred_element_type=_f32)
    h_ref[...] = h
    sd_ref[...], ss_ref[...] = _proj(h, att_ref)


def _tc_combine(aggr, den3, b, w, att):
    return pl.pallas_call(
        _tc_combine_body,
        out_shape=[
            jax.ShapeDtypeStruct((NPAD, D), _f32),
            jax.ShapeDtypeStruct((NPAD, 1), _f32),
            jax.ShapeDtypeStruct((NPAD, 1), _f32),
        ],
    )(aggr, den3, b, w, att)


def _tc_final_body(ag_ref, den_ref, b_ref, batch_ref, wf_ref, bf_ref, y_ref):
    a = ag_ref[0] + ag_ref[1]
    dsum = den_ref[...]
    h = jnp.maximum(a / (dsum + 1e-16) + b_ref[...][None, :], 0.0)
    ids = lax.broadcasted_iota(_i32, (1, NB), 1)
    oh = (batch_ref[...] == ids).astype(_f32)
    pooled = lax.dot_general(oh, h, (((0,), (0,)), ((), ())),
                             preferred_element_type=_f32)
    y_ref[...] = jnp.dot(pooled, wf_ref[...], preferred_element_type=_f32) + bf_ref[0]


def _tc_final(aggr, den3, b, batchcol, wf, bf):
    return pl.pallas_call(
        _tc_final_body,
        out_shape=jax.ShapeDtypeStruct((NB, 1), _f32),
    )(aggr, den3, b, batchcol, wf, bf)


def _lane_bcast(v, lane):
    idx = jnp.full((16, 1), lane, _i32)
    dnums = lax.GatherDimensionNumbers(offset_dims=(), collapsed_slice_dims=(0,),
                                       start_index_map=(0,))
    return lax.gather(v, idx, dnums, (1,),
                      mode=lax.GatherScatterMode.PROMISE_IN_BOUNDS)


def _sc_edge_body(edata_hbm, attv_hbm, sd_hbm, ss_hbm, h_hbm,
                  zrow_hbm,
                  aggr_out, den_out,
                  sd_v, ss_v, attv_v, edv, den_v, rows,
                  aggr_sh):
    cid = lax.axis_index("c")
    sid = lax.axis_index("s")
    wid = cid * 16 + sid

    pltpu.sync_copy(zrow_hbm, aggr_sh.at[pl.ds(sid * ROWS_PER_TILE, ROWS_PER_TILE)])
    pltpu.sync_copy(zrow_hbm.at[pl.ds(0, NPAD // D)], den_v)

    pltpu.sync_copy(sd_hbm.at[pl.ds(0, NPAD - D)], sd_v)
    pltpu.sync_copy(ss_hbm.at[pl.ds(0, NPAD - D)], ss_v)
    pltpu.sync_copy(attv_hbm, attv_v)
    plsc.subcore_barrier()

    def block_body(blk, carry):
        pltpu.sync_copy(edata_hbm.at[wid * (NCHUNK // EBLK) + blk], edv)
        atte = attv_v[pl.ds(0, 16)]
        for j in range(EBLK):
            c = blk * EBLK + j
            base = (wid * NCHUNK + c) * CH
            dv = edv.at[j * 3 + 1]
            pltpu.sync_copy(h_hbm.at[edv.at[j * 3]], rows)

            def group_body(g, carry2, _j=j, _base=base):
                sl = pl.ds(g * 16, 16)
                si = edv[_j * 3, sl]
                di = edv[_j * 3 + 1, sl]
                eab = plsc.bitcast(edv[_j * 3 + 2, sl], _f32)
                svals = plsc.load_gather(ss_v, [si])
                dvals = plsc.load_gather(sd_v, [di])
                alpha = svals + dvals + eab * atte
                alpha = jnp.where(alpha >= 0.0, alpha, 0.2 * alpha)
                gidx = _base + g * 16 + lax.iota(_i32, 16)
                keep = (si != di) | (gidx >= E)
                valid = gidx < EPRIME
                mf = jnp.where(keep & valid, 1.0, 0.0).astype(_f32)
                e16 = mf * jnp.exp(alpha)
                plsc.addupdate_scatter(
                    den_v, [jax.lax.shift_right_logical(di, 7),
                            jax.lax.bitwise_and(di, 127)], e16)
                for r in range(16):
                    eb = _lane_bcast(e16, r)
                    gr = g * 16 + r
                    for cc in range(D // 16):
                        csl = pl.ds(cc * 16, 16)
                        rows[gr, csl] = rows[gr, csl] * eb
                return carry2

            lax.fori_loop(0, NGRP, group_body, 0)
            pltpu.sync_copy(rows, aggr_sh.at[dv], add=True)
        return carry

    lax.fori_loop(0, NCHUNK // EBLK, block_body, 0)

    pltpu.sync_copy(den_v, den_out.at[wid])
    plsc.subcore_barrier()
    sl_rows = pl.ds(sid * ROWS_PER_TILE, ROWS_PER_TILE)
    pltpu.sync_copy(aggr_sh.at[sl_rows], aggr_out.at[cid, sl_rows])


def _sc_edge(edata, attv, sd, ss, h, zrow):
    mesh = plsc.VectorSubcoreMesh(core_axis_name="c", subcore_axis_name="s",
                                  num_cores=2, num_subcores=16)
    fn = pl.kernel(
        _sc_edge_body,
        out_type=(
            jax.ShapeDtypeStruct((2, NPAD, D), _f32),
            jax.ShapeDtypeStruct((NTILES, NPAD // D, D), _f32),
        ),
        mesh=mesh,
        compiler_params=pltpu.CompilerParams(needs_layout_passes=False),
        scratch_types=[
            pltpu.VMEM((NPAD - D,), _f32),
            pltpu.VMEM((NPAD - D,), _f32),
            pltpu.VMEM((16,), _f32),
            pltpu.VMEM((EBLK * 3, CH), _i32),
            pltpu.VMEM((NPAD // D, D), _f32),
            pltpu.VMEM((CH, D), _f32),
            pltpu.VMEM_SHARED((NPAD, D), _f32),
        ],
    )
    return fn(edata, attv, sd, ss, h, zrow)


def kernel(x, edge_index, edge_attr, batch, W0, att0, b0, W1, att1, b1, Wf, bf):
    loop = jnp.arange(N, dtype=_i32)
    pad_e = jnp.zeros((EARR - EPRIME,), _i32)
    src = jnp.concatenate([edge_index[0], loop, pad_e])
    dst = jnp.concatenate([edge_index[1], loop, pad_e])
    ea = jnp.concatenate([edge_attr, jnp.zeros((N + EARR - EPRIME,), _f32)])
    eabits = lax.bitcast_convert_type(ea, _i32)

    nchunks_total = EARR // CH
    edata = jnp.stack([src.reshape(nchunks_total, CH),
                       dst.reshape(nchunks_total, CH),
                       eabits.reshape(nchunks_total, CH)],
                      axis=1).reshape(nchunks_total // EBLK, EBLK * 3, CH)

    attv0 = jnp.broadcast_to(att0[0, 0, 2 * H:2 * H + 1], (16,)).astype(_f32)
    attv1 = jnp.broadcast_to(att1[0, 0, 2 * H:2 * H + 1], (16,)).astype(_f32)

    x_pad = jnp.pad(x, ((0, NPAD - N), (0, 0)))
    batchcol = jnp.concatenate(
        [batch.astype(_i32), jnp.full((NPAD - N,), NB, _i32)]).reshape(NPAD, 1)

    zrow = jnp.zeros((ROWS_PER_TILE, D), _f32)

    h0, sd0, ss0 = _tc_feats(x_pad, W0, att0)
    aggr0, den0 = _sc_edge(edata, attv0, sd0.reshape(NPAD), ss0.reshape(NPAD),
                           h0, zrow)

    h1, sd1, ss1 = _tc_combine(aggr0, den0.sum(axis=0).reshape(NPAD, 1), b0, W1, att1)
    aggr1, den1 = _sc_edge(edata, attv1, sd1.reshape(NPAD), ss1.reshape(NPAD),
                           h1, zrow)

    y = _tc_final(aggr1, den1.sum(axis=0).reshape(NPAD, 1), b1, batchcol, Wf, bf)
    return y.reshape(NB)
